# Initial kernel scaffold; baseline (speedup 1.0000x reference)
import jax
import jax.numpy as jnp
from jax.experimental import pallas as pl


def kernel(input_ids, W, complement_map):
    raise NotImplementedError("write your pallas kernel here")


# speedup vs baseline: 4.6634x; 4.6634x over previous
import functools

import jax
import jax.numpy as jnp
from jax import lax
from jax.experimental import pallas as pl
from jax.experimental.pallas import tpu as pltpu
from jax.experimental.pallas import tpu_sc as plsc

_NC = 2
_NS = 16
_CH = 128


def _build_table_body(cm_ref, w_ref, t_ref):
    Wm = w_ref[:]
    V, D = Wm.shape
    cm = cm_ref[:]
    vv = lax.broadcasted_iota(jnp.int32, (V, V), 1)
    onehot = (cm == vv).astype(jnp.float32)
    wrc = jnp.dot(onehot, Wm, preferred_element_type=jnp.float32,
                  precision=lax.Precision.HIGHEST)
    ii = lax.broadcasted_iota(jnp.int32, (D, D), 0)
    jj = lax.broadcasted_iota(jnp.int32, (D, D), 1)
    rev = (ii + jj == D - 1).astype(jnp.float32)
    t_ref[:, 0:D] = Wm
    t_ref[:, D:2 * D] = jnp.dot(wrc, rev, preferred_element_type=jnp.float32,
                                precision=lax.Precision.HIGHEST)


def kernel(input_ids, W, complement_map):
    Bb, S = input_ids.shape
    V, D = W.shape
    NT = Bb * S
    NW = _NC * _NS
    TPW = NT // NW
    NCH = TPW // _CH

    table = pl.pallas_call(
        _build_table_body,
        out_shape=jax.ShapeDtypeStruct((V, 2 * D), jnp.float32),
    )(complement_map.reshape(V, 1), W)

    ids2 = input_ids.reshape(NT // _CH, _CH)

    mesh = plsc.VectorSubcoreMesh(
        core_axis_name="c", subcore_axis_name="s",
        num_cores=_NC, num_subcores=_NS)

    @functools.partial(
        pl.kernel,
        out_type=jax.ShapeDtypeStruct((NT, 2 * D), jnp.float32),
        mesh=mesh,
        scratch_types=[
            pltpu.VMEM((NCH, _CH), jnp.int32),
            pltpu.VMEM((_CH, 2 * D), jnp.float32),
            pltpu.VMEM((_CH, 2 * D), jnp.float32),
            pltpu.SemaphoreType.DMA,
            pltpu.SemaphoreType.DMA,
            pltpu.SemaphoreType.DMA,
            pltpu.SemaphoreType.DMA,
        ],
    )
    def sc_embed(t_hbm, ids_hbm, out_hbm, ids_v, buf0, buf1, g0, g1, s0, s1):
        c = lax.axis_index("c")
        sb = lax.axis_index("s")
        wid = sb * _NC + c
        row0 = wid * NCH
        base = wid * TPW
        pltpu.sync_copy(ids_hbm.at[pl.ds(row0, NCH)], ids_v)

        bufs = (buf0, buf1)
        gsem = (g0, g1)
        ssem = (s0, s1)

        def start_gather(g):
            return pltpu.async_copy(
                t_hbm.at[ids_v.at[g]], bufs[g % 2], gsem[g % 2])

        def start_scatter(g):
            return pltpu.async_copy(
                bufs[g % 2], out_hbm.at[pl.ds(base + g * _CH, _CH)],
                ssem[g % 2])

        gathers = [None] * NCH
        scatters = [None] * NCH
        gathers[0] = start_gather(0)
        if NCH > 1:
            gathers[1] = start_gather(1)
        for g in range(NCH):
            gathers[g].wait()
            scatters[g] = start_scatter(g)
            if g + 2 < NCH:
                scatters[g].wait()
                gathers[g + 2] = start_gather(g + 2)
        for g in range(max(0, NCH - 2), NCH):
            scatters[g].wait()

    out = sc_embed(table, ids2)
    return out.reshape(Bb, S, 2 * D)
